# Initial kernel scaffold; baseline (speedup 1.0000x reference)
import jax
import jax.numpy as jnp
from jax.experimental import pallas as pl


def kernel(h, d, gate_W, gate_b, edge_index):
    raise NotImplementedError("write your pallas kernel here")


# speedup vs baseline: 17.8490x; 17.8490x over previous
import functools

import jax
import jax.numpy as jnp
from jax import lax
from jax.experimental import pallas as pl
from jax.experimental.pallas import tpu as pltpu
from jax.experimental.pallas import tpu_sc as plsc

N = 10000
E = 320000
D = 128

NC = 2
NS = 16
L = 16
NW = NC * NS
EPT = E // NW
K = 80
NCH = EPT // K
G = K // L
RA = 624
REM = N - NS * RA


def _gates_body(h_ref, w_ref, b_ref, g_ref):
    h = h_ref[...]
    w = w_ref[...]
    w2 = jnp.stack([w[0, :D], w[0, D:]], axis=1)
    bvec = jnp.concatenate([b_ref[...], jnp.zeros((1,), jnp.float32)])
    g_ref[...] = h @ w2 + bvec[None, :]


def _gates(h, gate_W, gate_b):
    g = pl.pallas_call(
        _gates_body,
        out_shape=jax.ShapeDtypeStruct((N, 2), jnp.float32),
    )(h, gate_W, gate_b)
    return g[:, 0], g[:, 1]


def _add_body(p_ref, z_ref):
    z_ref[...] = p_ref[0] + p_ref[1]


def _add_partials(partials):
    blk = 2000
    return pl.pallas_call(
        _add_body,
        grid=(N // blk,),
        in_specs=[pl.BlockSpec((NC, blk, D), lambda i: (0, i, 0))],
        out_specs=pl.BlockSpec((blk, D), lambda i: (i, 0)),
        out_shape=jax.ShapeDtypeStruct((N, D), jnp.float32),
    )(partials)


def _edge_e(sidx, didx, g1v, g2v, dv):
    ga = plsc.load_gather(g1v, [didx])
    gb = plsc.load_gather(g2v, [sidx])
    dd = plsc.load_gather(dv, [didx])
    ds_ = plsc.load_gather(dv, [sidx])
    x = ga + gb
    ax = jnp.abs(x)
    t = 1.0 - 2.0 / (jnp.exp(2.0 * ax) + 1.0)
    t = jnp.where(x < 0.0, -t, t)
    return t * dd * ds_


def _sc_gate_kernel(src_hbm, dst_hbm, g1_hbm, g2_hbm, d_hbm, e_hbm,
                    srcv, dstv, g1v, g2v, dv, ebuf):
    c = lax.axis_index("c")
    s = lax.axis_index("s")
    wid = c * NS + s

    pltpu.sync_copy(g1_hbm, g1v)
    pltpu.sync_copy(g2_hbm, g2v)
    pltpu.sync_copy(d_hbm, dv)
    pltpu.sync_copy(src_hbm.at[wid], srcv)
    pltpu.sync_copy(dst_hbm.at[wid], dstv)

    def body(j, carry):
        for g in range(G):
            sidx = srcv[j, pl.ds(g * L, L)]
            didx = dstv[j, pl.ds(g * L, L)]
            ebuf[j, pl.ds(g * L, L)] = _edge_e(sidx, didx, g1v, g2v, dv)
        return carry
    lax.fori_loop(0, NCH, body, 0)

    pltpu.sync_copy(ebuf, e_hbm.at[wid])


@functools.cache
def _sc_gate():
    return pl.kernel(
        _sc_gate_kernel,
        out_type=jax.ShapeDtypeStruct((NW, NCH, K), jnp.float32),
        mesh=plsc.VectorSubcoreMesh(core_axis_name="c", subcore_axis_name="s"),
        scratch_types=[
            pltpu.VMEM((NCH, K), jnp.int32),
            pltpu.VMEM((NCH, K), jnp.int32),
            pltpu.VMEM((N,), jnp.float32),
            pltpu.VMEM((N,), jnp.float32),
            pltpu.VMEM((N,), jnp.float32),
            pltpu.VMEM((NCH, K), jnp.float32),
        ],
        compiler_params=pltpu.CompilerParams(needs_layout_passes=False),
    )


def _sc_push_kernel(src_hbm, dst_hbm, e_hbm, h_hbm, out_hbm,
                    srcb, dstb, eb, gbuf0, sbuf0, zsh, gsem0, ssem0, msem0):
    c = lax.axis_index("c")
    s = lax.axis_index("s")
    wid = c * NS + s

    def zrow(i, carry):
        for m in range(D // L):
            gbuf0[i, pl.ds(m * L, L)] = jnp.zeros((L,), jnp.float32)
        return carry
    lax.fori_loop(0, K, zrow, 0)
    for t in range(RA // K):
        pltpu.sync_copy(gbuf0, zsh.at[pl.ds(s * RA + t * K, K)])
    rem0 = RA - (RA // K) * K
    if rem0:
        pltpu.sync_copy(gbuf0.at[pl.ds(0, rem0)],
                        zsh.at[pl.ds(s * RA + (RA // K) * K, rem0)])

    @pl.when(s == 0)
    def _zero_rem():
        pltpu.sync_copy(gbuf0.at[pl.ds(0, REM)], zsh.at[pl.ds(NS * RA, REM)])

    plsc.subcore_barrier()

    def body(j, carry):
        a = pltpu.async_copy(src_hbm.at[wid, j], srcb.at[0], msem0)
        b = pltpu.async_copy(dst_hbm.at[wid, j], dstb.at[0], msem0)
        ce = pltpu.async_copy(e_hbm.at[wid, j], eb.at[0], msem0)
        a.wait()
        b.wait()
        ce.wait()
        r = pltpu.async_copy(h_hbm.at[srcb.at[0]], gbuf0, gsem0)
        r.wait()
        gdn = lax.GatherDimensionNumbers(
            offset_dims=(), collapsed_slice_dims=(0,), start_index_map=(0,))
        for g in range(G):
            evec = eb[0, pl.ds(g * L, L)]
            for k in range(L):
                scale = lax.gather(
                    evec, jnp.full((L, 1), k, jnp.int32), gdn, (1,),
                    mode=lax.GatherScatterMode.PROMISE_IN_BOUNDS)
                r_ = g * L + k
                for m in range(D // L):
                    sbuf0[r_, pl.ds(m * L, L)] = (
                        gbuf0[r_, pl.ds(m * L, L)] * scale)
        pltpu.async_copy(sbuf0, zsh.at[dstb.at[0]], ssem0,
                         add=True).wait()
        return carry
    lax.fori_loop(0, NCH, body, 0)

    plsc.subcore_barrier()
    pltpu.sync_copy(zsh.at[pl.ds(s * RA, RA)],
                    out_hbm.at[c, pl.ds(s * RA, RA)])

    @pl.when(s == 0)
    def _copy_rem():
        pltpu.sync_copy(zsh.at[pl.ds(NS * RA, REM)],
                        out_hbm.at[c, pl.ds(NS * RA, REM)])


@functools.cache
def _sc_push():
    return pl.kernel(
        _sc_push_kernel,
        out_type=jax.ShapeDtypeStruct((NC, N, D), jnp.float32),
        mesh=plsc.VectorSubcoreMesh(core_axis_name="c", subcore_axis_name="s"),
        scratch_types=[
            pltpu.VMEM((2, K), jnp.int32),
            pltpu.VMEM((2, K), jnp.int32),
            pltpu.VMEM((2, K), jnp.float32),
            pltpu.VMEM((K, D), jnp.float32),
            pltpu.VMEM((K, D), jnp.float32),
            pltpu.VMEM_SHARED((N, D), jnp.float32),
            pltpu.SemaphoreType.DMA,
            pltpu.SemaphoreType.DMA,
            pltpu.SemaphoreType.DMA,
        ],
        compiler_params=pltpu.CompilerParams(needs_layout_passes=False),
    )


def kernel(h, d, gate_W, gate_b, edge_index):
    g1, g2 = _gates(h, gate_W, gate_b)
    src = edge_index[0].reshape(NW, NCH, K)
    dst = edge_index[1].reshape(NW, NCH, K)
    e = _sc_gate()(src, dst, g1, g2, d)
    partials = _sc_push()(src, dst, e, h)
    return _add_partials(partials)


# speedup vs baseline: 20.3086x; 1.1378x over previous
import functools

import jax
import jax.numpy as jnp
from jax import lax
from jax.experimental import pallas as pl
from jax.experimental.pallas import tpu as pltpu
from jax.experimental.pallas import tpu_sc as plsc

N = 10000
E = 320000
D = 128

NC = 2
NS = 16
L = 16
NW = NC * NS
EPT = E // NW
K = 80
NCH = EPT // K
G = K // L
RA = 624
REM = N - NS * RA


def _gates_body(h_ref, w_ref, b_ref, g_ref):
    h = h_ref[...]
    w = w_ref[...]
    w2 = jnp.stack([w[0, :D], w[0, D:]], axis=1)
    bvec = jnp.concatenate([b_ref[...], jnp.zeros((1,), jnp.float32)])
    g_ref[...] = h @ w2 + bvec[None, :]


def _gates(h, gate_W, gate_b):
    g = pl.pallas_call(
        _gates_body,
        out_shape=jax.ShapeDtypeStruct((N, 2), jnp.float32),
    )(h, gate_W, gate_b)
    return g[:, 0], g[:, 1]


def _add_body(p_ref, z_ref):
    z_ref[...] = p_ref[0] + p_ref[1]


def _add_partials(partials):
    blk = 2000
    return pl.pallas_call(
        _add_body,
        grid=(N // blk,),
        in_specs=[pl.BlockSpec((NC, blk, D), lambda i: (0, i, 0))],
        out_specs=pl.BlockSpec((blk, D), lambda i: (i, 0)),
        out_shape=jax.ShapeDtypeStruct((N, D), jnp.float32),
    )(partials)


def _edge_e(sidx, didx, g1v, g2v, dv):
    ga = plsc.load_gather(g1v, [didx])
    gb = plsc.load_gather(g2v, [sidx])
    dd = plsc.load_gather(dv, [didx])
    ds_ = plsc.load_gather(dv, [sidx])
    x = ga + gb
    ax = jnp.abs(x)
    t = 1.0 - 2.0 / (jnp.exp(2.0 * ax) + 1.0)
    t = jnp.where(x < 0.0, -t, t)
    return t * dd * ds_


def _sc_gate_kernel(src_hbm, dst_hbm, g1_hbm, g2_hbm, d_hbm, e_hbm,
                    srcv, dstv, g1v, g2v, dv, ebuf):
    c = lax.axis_index("c")
    s = lax.axis_index("s")
    wid = c * NS + s

    pltpu.sync_copy(g1_hbm, g1v)
    pltpu.sync_copy(g2_hbm, g2v)
    pltpu.sync_copy(d_hbm, dv)
    pltpu.sync_copy(src_hbm.at[wid], srcv)
    pltpu.sync_copy(dst_hbm.at[wid], dstv)

    def body(j, carry):
        for g in range(G):
            sidx = srcv[j, pl.ds(g * L, L)]
            didx = dstv[j, pl.ds(g * L, L)]
            ebuf[j, pl.ds(g * L, L)] = _edge_e(sidx, didx, g1v, g2v, dv)
        return carry
    lax.fori_loop(0, NCH, body, 0)

    pltpu.sync_copy(ebuf, e_hbm.at[wid])


@functools.cache
def _sc_gate():
    return pl.kernel(
        _sc_gate_kernel,
        out_type=jax.ShapeDtypeStruct((NW, NCH, K), jnp.float32),
        mesh=plsc.VectorSubcoreMesh(core_axis_name="c", subcore_axis_name="s"),
        scratch_types=[
            pltpu.VMEM((NCH, K), jnp.int32),
            pltpu.VMEM((NCH, K), jnp.int32),
            pltpu.VMEM((N,), jnp.float32),
            pltpu.VMEM((N,), jnp.float32),
            pltpu.VMEM((N,), jnp.float32),
            pltpu.VMEM((NCH, K), jnp.float32),
        ],
        compiler_params=pltpu.CompilerParams(needs_layout_passes=False),
    )


KC = 16
NKC = EPT // KC
_GDN = lax.GatherDimensionNumbers(
    offset_dims=(), collapsed_slice_dims=(0,), start_index_map=(0,))


def _sc_push_kernel(src_hbm, dst_hbm, e_hbm, h_hbm, out_hbm,
                    srcv, dstv, ev, gbufA, gbufB, sbufA, sbufB, zsh,
                    gsemA, gsemB, ssemA, ssemB):
    c = lax.axis_index("c")
    s = lax.axis_index("s")
    wid = c * NS + s

    for i in range(KC):
        for m in range(D // L):
            sbufA[i, pl.ds(m * L, L)] = jnp.zeros((L,), jnp.float32)
    for t in range(RA // KC):
        pltpu.sync_copy(sbufA, zsh.at[pl.ds(s * RA + t * KC, KC)])

    @pl.when(s == 0)
    def _zero_rem():
        pltpu.sync_copy(sbufA.at[pl.ds(0, REM)], zsh.at[pl.ds(NS * RA, REM)])

    pltpu.sync_copy(src_hbm.at[wid], srcv)
    pltpu.sync_copy(dst_hbm.at[wid], dstv)
    pltpu.sync_copy(e_hbm.at[wid], ev)

    plsc.subcore_barrier()

    def fire_gather(q, gbuf, gsem):
        return pltpu.async_copy(h_hbm.at[srcv.at[pl.ds(q * KC, KC)]],
                                gbuf, gsem)

    def wait_gather(gbuf, gsem):
        pltpu.make_async_copy(h_hbm.at[srcv.at[pl.ds(0, KC)]],
                              gbuf, gsem).wait()

    def do_chunk(j, gbuf, sbuf, ssem):
        evec = ev[pl.ds(j * KC, KC)]
        for k in range(KC):
            scale = lax.gather(
                evec, jnp.full((L, 1), k, jnp.int32), _GDN, (1,),
                mode=lax.GatherScatterMode.PROMISE_IN_BOUNDS)
            for m in range(D // L):
                sbuf[k, pl.ds(m * L, L)] = gbuf[k, pl.ds(m * L, L)] * scale
        didx = dstv[pl.ds(j * KC, KC)]
        return pltpu.async_copy(sbuf, zsh.at[didx], ssem, add=True)

    def wait_scatter(sbuf, ssem):
        pltpu.make_async_copy(sbuf, zsh.at[pl.ds(0, KC)], ssem).wait()

    fire_gather(0, gbufA, gsemA)
    fire_gather(1, gbufB, gsemB)

    def body(i, carry):
        wait_gather(gbufA, gsemA)

        @pl.when(i > 0)
        def _wa():
            wait_scatter(sbufA, ssemA)
        do_chunk(2 * i, gbufA, sbufA, ssemA)
        fire_gather(2 * i + 2, gbufA, gsemA)
        wait_gather(gbufB, gsemB)

        @pl.when(i > 0)
        def _wb():
            wait_scatter(sbufB, ssemB)
        do_chunk(2 * i + 1, gbufB, sbufB, ssemB)

        @pl.when(i < NKC // 2 - 1)
        def _fg():
            fire_gather(2 * i + 3, gbufB, gsemB)
        return carry
    lax.fori_loop(0, NKC // 2, body, 0)

    wait_gather(gbufA, gsemA)
    wait_scatter(sbufA, ssemA)
    do_chunk(NKC - 1, gbufA, sbufA, ssemA)
    wait_scatter(sbufA, ssemA)
    wait_scatter(sbufB, ssemB)

    plsc.subcore_barrier()
    pltpu.sync_copy(zsh.at[pl.ds(s * RA, RA)],
                    out_hbm.at[c, pl.ds(s * RA, RA)])

    @pl.when(s == 0)
    def _copy_rem():
        pltpu.sync_copy(zsh.at[pl.ds(NS * RA, REM)],
                        out_hbm.at[c, pl.ds(NS * RA, REM)])


@functools.cache
def _sc_push():
    return pl.kernel(
        _sc_push_kernel,
        out_type=jax.ShapeDtypeStruct((NC, N, D), jnp.float32),
        mesh=plsc.VectorSubcoreMesh(core_axis_name="c", subcore_axis_name="s"),
        scratch_types=[
            pltpu.VMEM((EPT,), jnp.int32),
            pltpu.VMEM((EPT,), jnp.int32),
            pltpu.VMEM((EPT,), jnp.float32),
            pltpu.VMEM((KC, D), jnp.float32),
            pltpu.VMEM((KC, D), jnp.float32),
            pltpu.VMEM((KC, D), jnp.float32),
            pltpu.VMEM((KC, D), jnp.float32),
            pltpu.VMEM_SHARED((N, D), jnp.float32),
            pltpu.SemaphoreType.DMA,
            pltpu.SemaphoreType.DMA,
            pltpu.SemaphoreType.DMA,
            pltpu.SemaphoreType.DMA,
        ],
        compiler_params=pltpu.CompilerParams(needs_layout_passes=False),
    )


def kernel(h, d, gate_W, gate_b, edge_index):
    g1, g2 = _gates(h, gate_W, gate_b)
    src = edge_index[0].reshape(NW, NCH, K)
    dst = edge_index[1].reshape(NW, NCH, K)
    e = _sc_gate()(src, dst, g1, g2, d)
    partials = _sc_push()(edge_index[0].reshape(NW, EPT),
                          edge_index[1].reshape(NW, EPT),
                          e.reshape(NW, EPT), h)
    return _add_partials(partials)


# speedup vs baseline: 26.9167x; 1.3254x over previous
import functools

import jax
import jax.numpy as jnp
from jax import lax
from jax.experimental import pallas as pl
from jax.experimental.pallas import tpu as pltpu
from jax.experimental.pallas import tpu_sc as plsc

N = 10000
E = 320000
D = 128

NC = 2
NS = 16
L = 16
NW = NC * NS
EPT = E // NW
K = 80
NCH = EPT // K
G = K // L
RA = 624
REM = N - NS * RA


def _gates_body(h_ref, w_ref, b_ref, g_ref):
    h = h_ref[...]
    w = w_ref[...]
    w2 = jnp.stack([w[0, :D], w[0, D:]], axis=1)
    bvec = jnp.concatenate([b_ref[...], jnp.zeros((1,), jnp.float32)])
    g_ref[...] = h @ w2 + bvec[None, :]


def _gates(h, gate_W, gate_b):
    g = pl.pallas_call(
        _gates_body,
        out_shape=jax.ShapeDtypeStruct((N, 2), jnp.float32),
    )(h, gate_W, gate_b)
    return g[:, 0], g[:, 1]


def _add_body(p_ref, z_ref):
    z_ref[...] = p_ref[0] + p_ref[1]


def _add_partials(partials):
    blk = 2000
    return pl.pallas_call(
        _add_body,
        grid=(N // blk,),
        in_specs=[pl.BlockSpec((NC, blk, D), lambda i: (0, i, 0))],
        out_specs=pl.BlockSpec((blk, D), lambda i: (i, 0)),
        out_shape=jax.ShapeDtypeStruct((N, D), jnp.float32),
    )(partials)


def _edge_e(sidx, didx, g1v, g2v, dv):
    ga = plsc.load_gather(g1v, [didx])
    gb = plsc.load_gather(g2v, [sidx])
    dd = plsc.load_gather(dv, [didx])
    ds_ = plsc.load_gather(dv, [sidx])
    x = ga + gb
    ax = jnp.abs(x)
    t = 1.0 - 2.0 / (jnp.exp(2.0 * ax) + 1.0)
    t = jnp.where(x < 0.0, -t, t)
    return t * dd * ds_


def _sc_gate_kernel(src_hbm, dst_hbm, g1_hbm, g2_hbm, d_hbm, e_hbm,
                    srcv, dstv, g1v, g2v, dv, ebuf):
    c = lax.axis_index("c")
    s = lax.axis_index("s")
    wid = c * NS + s

    pltpu.sync_copy(g1_hbm, g1v)
    pltpu.sync_copy(g2_hbm, g2v)
    pltpu.sync_copy(d_hbm, dv)
    pltpu.sync_copy(src_hbm.at[wid], srcv)
    pltpu.sync_copy(dst_hbm.at[wid], dstv)

    def body(j, carry):
        for g in range(G):
            sidx = srcv[j, pl.ds(g * L, L)]
            didx = dstv[j, pl.ds(g * L, L)]
            ebuf[j, pl.ds(g * L, L)] = _edge_e(sidx, didx, g1v, g2v, dv)
        return carry
    lax.fori_loop(0, NCH, body, 0)

    pltpu.sync_copy(ebuf, e_hbm.at[wid])


@functools.cache
def _sc_gate():
    return pl.kernel(
        _sc_gate_kernel,
        out_type=jax.ShapeDtypeStruct((NW, NCH, K), jnp.float32),
        mesh=plsc.VectorSubcoreMesh(core_axis_name="c", subcore_axis_name="s"),
        scratch_types=[
            pltpu.VMEM((NCH, K), jnp.int32),
            pltpu.VMEM((NCH, K), jnp.int32),
            pltpu.VMEM((N,), jnp.float32),
            pltpu.VMEM((N,), jnp.float32),
            pltpu.VMEM((N,), jnp.float32),
            pltpu.VMEM((NCH, K), jnp.float32),
        ],
        compiler_params=pltpu.CompilerParams(needs_layout_passes=False),
    )


GC = 32
NGC = 312
_GDN = lax.GatherDimensionNumbers(
    offset_dims=(), collapsed_slice_dims=(0,), start_index_map=(0,))


def _sc_push_kernel(src_hbm, dst_hbm, e_hbm, h_hbm, out_hbm,
                    srcv, dstv, ev, gbufA, gbufB,
                    sbufA0, sbufA1, sbufB0, sbufB1, zsh,
                    gsemA, gsemB, ssemA0, ssemA1, ssemB0, ssemB1):
    c = lax.axis_index("c")
    s = lax.axis_index("s")
    wid = c * NS + s

    for i in range(L):
        for m in range(D // L):
            sbufA0[i, pl.ds(m * L, L)] = jnp.zeros((L,), jnp.float32)
    for t in range(RA // L):
        pltpu.sync_copy(sbufA0, zsh.at[pl.ds(s * RA + t * L, L)])

    @pl.when(s == 0)
    def _zero_rem():
        pltpu.sync_copy(sbufA0, zsh.at[pl.ds(NS * RA, REM)])

    pltpu.sync_copy(src_hbm.at[wid], srcv)
    pltpu.sync_copy(dst_hbm.at[wid], dstv)
    pltpu.sync_copy(e_hbm.at[wid], ev)

    plsc.subcore_barrier()

    def fire_gather(q, nrows, gbuf, gsem):
        return pltpu.async_copy(h_hbm.at[srcv.at[pl.ds(q * GC, nrows)]],
                                gbuf.at[pl.ds(0, nrows)], gsem)

    def wait_gather(nrows, gbuf, gsem):
        pltpu.make_async_copy(h_hbm.at[srcv.at[pl.ds(0, nrows)]],
                              gbuf.at[pl.ds(0, nrows)], gsem).wait()

    def do_sub(base, gbuf, goff, sbuf, ssem):
        evec = ev[pl.ds(base, L)]
        for k in range(L):
            scale = lax.gather(
                evec, jnp.full((L, 1), k, jnp.int32), _GDN, (1,),
                mode=lax.GatherScatterMode.PROMISE_IN_BOUNDS)
            for m in range(D // L):
                sbuf[k, pl.ds(m * L, L)] = (
                    gbuf[goff + k, pl.ds(m * L, L)] * scale)
        didx = dstv[pl.ds(base, L)]
        return pltpu.async_copy(sbuf, zsh.at[didx], ssem, add=True)

    def wait_scatter(sbuf, ssem):
        pltpu.make_async_copy(sbuf, zsh.at[pl.ds(0, L)], ssem).wait()

    fire_gather(0, GC, gbufA, gsemA)
    fire_gather(1, GC, gbufB, gsemB)

    def body(i, carry):
        wait_gather(GC, gbufA, gsemA)

        @pl.when(i > 0)
        def _wa():
            wait_scatter(sbufA0, ssemA0)
            wait_scatter(sbufA1, ssemA1)
        do_sub(2 * i * GC, gbufA, 0, sbufA0, ssemA0)
        do_sub(2 * i * GC + L, gbufA, L, sbufA1, ssemA1)

        @pl.when(i < NGC // 2 - 1)
        def _fga():
            fire_gather(2 * i + 2, GC, gbufA, gsemA)
        wait_gather(GC, gbufB, gsemB)

        @pl.when(i > 0)
        def _wb():
            wait_scatter(sbufB0, ssemB0)
            wait_scatter(sbufB1, ssemB1)
        do_sub((2 * i + 1) * GC, gbufB, 0, sbufB0, ssemB0)
        do_sub((2 * i + 1) * GC + L, gbufB, L, sbufB1, ssemB1)

        @pl.when(i < NGC // 2 - 1)
        def _fgb():
            fire_gather(2 * i + 3, GC, gbufB, gsemB)
        return carry
    lax.fori_loop(0, NGC // 2, body, 0)

    fire_gather(NGC, L, gbufA, gsemA).wait()
    wait_scatter(sbufA0, ssemA0)
    do_sub(NGC * GC, gbufA, 0, sbufA0, ssemA0)
    wait_scatter(sbufA0, ssemA0)
    wait_scatter(sbufA1, ssemA1)
    wait_scatter(sbufB0, ssemB0)
    wait_scatter(sbufB1, ssemB1)

    plsc.subcore_barrier()
    pltpu.sync_copy(zsh.at[pl.ds(s * RA, RA)],
                    out_hbm.at[c, pl.ds(s * RA, RA)])

    @pl.when(s == 0)
    def _copy_rem():
        pltpu.sync_copy(zsh.at[pl.ds(NS * RA, REM)],
                        out_hbm.at[c, pl.ds(NS * RA, REM)])


@functools.cache
def _sc_push():
    return pl.kernel(
        _sc_push_kernel,
        out_type=jax.ShapeDtypeStruct((NC, N, D), jnp.float32),
        mesh=plsc.VectorSubcoreMesh(core_axis_name="c", subcore_axis_name="s"),
        scratch_types=[
            pltpu.VMEM((EPT,), jnp.int32),
            pltpu.VMEM((EPT,), jnp.int32),
            pltpu.VMEM((EPT,), jnp.float32),
            pltpu.VMEM((GC, D), jnp.float32),
            pltpu.VMEM((GC, D), jnp.float32),
            pltpu.VMEM((L, D), jnp.float32),
            pltpu.VMEM((L, D), jnp.float32),
            pltpu.VMEM((L, D), jnp.float32),
            pltpu.VMEM((L, D), jnp.float32),
            pltpu.VMEM_SHARED((N, D), jnp.float32),
            pltpu.SemaphoreType.DMA,
            pltpu.SemaphoreType.DMA,
            pltpu.SemaphoreType.DMA,
            pltpu.SemaphoreType.DMA,
            pltpu.SemaphoreType.DMA,
            pltpu.SemaphoreType.DMA,
        ],
        compiler_params=pltpu.CompilerParams(needs_layout_passes=False),
    )


def kernel(h, d, gate_W, gate_b, edge_index):
    g1, g2 = _gates(h, gate_W, gate_b)
    src = edge_index[0].reshape(NW, NCH, K)
    dst = edge_index[1].reshape(NW, NCH, K)
    e = _sc_gate()(src, dst, g1, g2, d)
    partials = _sc_push()(edge_index[0].reshape(NW, EPT),
                          edge_index[1].reshape(NW, EPT),
                          e.reshape(NW, EPT), h)
    return _add_partials(partials)
